# baseline (speedup 1.0000x reference)
import functools

import jax
import jax.numpy as jnp
import numpy as np
from jax.experimental import pallas as pl
from jax.experimental.pallas import tpu as pltpu

N_NODE = 512
HIDDEN = 256
B = 64
NB = N_NODE // B
EPS = 1e-5

_PAIRS = [(i, j) for i in range(NB) for j in range(NB) if j >= i]
NSTEP = len(_PAIRS)


def _edge_scalars(ub, ib, P_ref, W1T_ref, R_ref, b2):
    A2 = P_ref[0:1, :].astype(jnp.bfloat16)
    C2 = P_ref[1:2, :].astype(jnp.bfloat16)
    be0 = P_ref[2:3, :].astype(jnp.bfloat16)
    b1 = P_ref[3:4, :].astype(jnp.bfloat16)
    g1 = P_ref[4:5, :].astype(jnp.bfloat16)
    be1 = P_ref[5:6, :].astype(jnp.bfloat16)
    w2 = P_ref[6:7, :].astype(jnp.bfloat16)

    a0 = ub * A2 + (ib * C2 + be0)
    x = a0 + a0 * jnp.tanh(a0)

    haug = jnp.dot(x, W1T_ref[...], preferred_element_type=jnp.float32)
    t1 = (haug[:, :HIDDEN] - haug[:, HIDDEN:HIDDEN + 1]).astype(jnp.bfloat16) + b1
    t1sq = t1 * t1
    v = jnp.dot(t1sq, R_ref[...], preferred_element_type=jnp.float32)[:, :1]
    v = v * jnp.float32(1.0 / HIDDEN)
    a1 = (t1 * jax.lax.rsqrt(v + EPS).astype(jnp.bfloat16)) * g1 + be1
    y = a1 + a1 * jnp.tanh(a1)

    s = jnp.dot(y, R_ref[...], preferred_element_type=jnp.float32)[:, 1] + b2
    return s


def _accumulate(F, I, J, pi4, pj4, fp_ref):
    rows = jax.lax.broadcasted_iota(jnp.int32, (B, B), 0) + I * B
    cols = jax.lax.broadcasted_iota(jnp.int32, (B, B), 1) + J * B
    F = jnp.where(rows == cols, 0.0, F)

    fp_ref[pl.ds(I * B, B), :] += jnp.dot(F, pj4,
                                          preferred_element_type=jnp.float32)

    @pl.when(J != I)
    def _mirror():
        ft = jax.lax.dot_general(F, pi4, (((0,), (0,)), ((), ())),
                                 preferred_element_type=jnp.float32)
        fp_ref[pl.ds(J * B, B), :] += ft


def _egnn_block(ia_ref, ja_ref, pia_ref, pja_ref, pib_ref, pjb_ref, pf_ref,
                P_ref, W1T_ref, R_ref, b2_ref, out_ref, fp_ref):
    p = pl.program_id(0)
    Ia = ia_ref[2 * p]
    Ja = ja_ref[2 * p]
    Ib = ia_ref[2 * p + 1]
    Jb = ja_ref[2 * p + 1]

    @pl.when(p == 0)
    def _init():
        fp_ref[...] = jnp.zeros_like(fp_ref)

    pia4 = pia_ref[...]
    pja4 = pja_ref[...]
    pib4 = pib_ref[...]
    pjb4 = pjb_ref[...]

    diffa = pia4[:, None, :3] - pja4[None, :, :3]
    diffb = pib4[:, None, :3] - pjb4[None, :, :3]
    ra = jnp.sum(diffa * diffa, axis=-1)
    rb = jnp.sum(diffb * diffb, axis=-1)

    mom = P_ref[7:8, :]
    va = mom[0, 0]
    cov = mom[0, 1]
    vc = mom[0, 2]

    def _scal(r):
        inv = jax.lax.rsqrt(r * r * va + 2.0 * r * cov + (vc + EPS))
        ub = (r * inv).reshape(B * B, 1).astype(jnp.bfloat16)
        ib = inv.reshape(B * B, 1).astype(jnp.bfloat16)
        return ub, ib

    uba, iba = _scal(ra)
    ubb, ibb = _scal(rb)
    ub = jnp.concatenate([uba, ubb], axis=0)
    ib = jnp.concatenate([iba, ibb], axis=0)

    s = _edge_scalars(ub, ib, P_ref, W1T_ref, R_ref, b2_ref[0, 0])
    Fall = s.reshape(2 * B, B)
    _accumulate(Fall[:B, :], Ia, Ja, pia4, pja4, fp_ref)
    _accumulate(Fall[B:, :], Ib, Jb, pib4, pjb4, fp_ref)

    @pl.when(p == NSTEP // 2 - 1)
    def _finalize():
        pf = pf_ref[:, :3]
        fp4 = fp_ref[...]
        rowsum = fp4[:, 3:4]
        fpos = fp4[:, :3]
        out_ref[...] = pf + (rowsum * pf - fpos) * (1.0 / (N_NODE - 1))


@functools.partial(jax.jit, static_argnames=())
def _egnn_call(pos4, P, W1T, R, b2):
    ia = jnp.asarray(np.array([p[0] for p in _PAIRS], np.int32))
    ja = jnp.asarray(np.array([p[1] for p in _PAIRS], np.int32))
    grid_spec = pltpu.PrefetchScalarGridSpec(
        num_scalar_prefetch=2,
        grid=(NSTEP // 2,),
        in_specs=[
            pl.BlockSpec((B, 4), lambda p, ia, ja: (ia[2 * p], 0)),
            pl.BlockSpec((B, 4), lambda p, ia, ja: (ja[2 * p], 0)),
            pl.BlockSpec((B, 4), lambda p, ia, ja: (ia[2 * p + 1], 0)),
            pl.BlockSpec((B, 4), lambda p, ia, ja: (ja[2 * p + 1], 0)),
            pl.BlockSpec((N_NODE, 4), lambda p, ia, ja: (0, 0)),
            pl.BlockSpec((8, HIDDEN), lambda p, ia, ja: (0, 0)),
            pl.BlockSpec((HIDDEN, HIDDEN + 128), lambda p, ia, ja: (0, 0)),
            pl.BlockSpec((HIDDEN, 128), lambda p, ia, ja: (0, 0)),
            pl.BlockSpec((1, 1), lambda p, ia, ja: (0, 0)),
        ],
        out_specs=pl.BlockSpec((N_NODE, 3), lambda p, ia, ja: (0, 0)),
        scratch_shapes=[pltpu.VMEM((N_NODE, 4), jnp.float32)],
    )
    return pl.pallas_call(
        _egnn_block,
        grid_spec=grid_spec,
        out_shape=jax.ShapeDtypeStruct((N_NODE, 3), jnp.float32),
    )(ia, ja, pos4, pos4, pos4, pos4, pos4, P, W1T, R, b2)


def kernel(pos, t, W0, b0, g0, be0, W1, b1, g1, be1, W2, b2,
           senders, receivers):
    A = W0[:, 0]
    C = t * W0[:, 1] + b0
    Am = A - jnp.mean(A)
    Cm = C - jnp.mean(C)
    va = jnp.mean(Am * Am)
    cov = jnp.mean(Am * Cm)
    vc = jnp.mean(Cm * Cm)
    mom = jnp.zeros((HIDDEN,), jnp.float32).at[0].set(va).at[1].set(cov).at[2].set(vc)
    P = jnp.stack([0.5 * Am * g0, 0.5 * Cm * g0, 0.5 * be0,
                   b1 - jnp.mean(b1), 0.5 * g1, 0.5 * be1, W2[0], mom])
    W1T = W1.T
    w1m = jnp.mean(W1T, axis=1, keepdims=True)
    W1Ta = jnp.concatenate(
        [W1T, w1m, jnp.zeros((HIDDEN, 127), jnp.float32)], axis=1
    ).astype(jnp.bfloat16)
    R = jnp.concatenate(
        [jnp.ones((HIDDEN, 1), jnp.float32), W2[0][:, None],
         jnp.zeros((HIDDEN, 126), jnp.float32)], axis=1).astype(jnp.bfloat16)
    b2r = b2.reshape(1, 1)
    pos4 = jnp.concatenate([pos, jnp.ones((N_NODE, 1), jnp.float32)], axis=1)
    return _egnn_call(pos4, P, W1Ta, R, b2r)


# speedup vs baseline: 1.1133x; 1.1133x over previous
import functools

import jax
import jax.numpy as jnp
import numpy as np
from jax.experimental import pallas as pl
from jax.experimental.pallas import tpu as pltpu

N_NODE = 512
HIDDEN = 256
B = 64
NB = N_NODE // B
EPS = 1e-5

_PAIRS = [(i, j) for i in range(NB) for j in range(NB) if j >= i]
NSTEP = len(_PAIRS)


def _edge_scalars(ub, ib, P_ref, W1T_ref, R_ref, b2):
    A2 = P_ref[0:1, :].astype(jnp.bfloat16)
    C2 = P_ref[1:2, :].astype(jnp.bfloat16)
    be0 = P_ref[2:3, :].astype(jnp.bfloat16)
    b1 = P_ref[3:4, :].astype(jnp.bfloat16)
    g1 = P_ref[4:5, :].astype(jnp.bfloat16)
    be1 = P_ref[5:6, :].astype(jnp.bfloat16)
    w2 = P_ref[6:7, :].astype(jnp.bfloat16)

    a0 = ub * A2 + (ib * C2 + be0)
    x = a0 + a0 * jnp.tanh(a0)

    haug = jnp.dot(x, W1T_ref[...], preferred_element_type=jnp.float32)
    t1 = (haug[:, :HIDDEN] - haug[:, HIDDEN:HIDDEN + 1]).astype(jnp.bfloat16) + b1
    t1sq = t1 * t1
    v = jnp.dot(t1sq, R_ref[...], preferred_element_type=jnp.float32)[:, :1]
    a1 = (t1 * jax.lax.rsqrt(v + EPS).astype(jnp.bfloat16)) * g1 + be1
    y = a1 + a1 * jnp.tanh(a1)

    s = jnp.dot(y, R_ref[...], preferred_element_type=jnp.float32)[:, 1] + b2
    return s


def _egnn_block(ia_ref, ja_ref, pi_ref, pj_ref, pf_ref, P_ref, W1T_ref,
                R_ref, b2_ref, out_ref, fp_ref):
    p = pl.program_id(0)
    I = ia_ref[p]
    J = ja_ref[p]

    @pl.when(p == 0)
    def _init():
        fp_ref[...] = jnp.zeros_like(fp_ref)

    pi4 = pi_ref[...]
    pj4 = pj_ref[...]
    pi = pi4[:, :3]
    pj = pj4[:, :3]

    diff = pi[:, None, :] - pj[None, :, :]
    r = jnp.sum(diff * diff, axis=-1)

    mom = P_ref[7:8, :]
    va = mom[0, 0]
    cov = mom[0, 1]
    vc = mom[0, 2]
    inv = jax.lax.rsqrt(r * r * va + 2.0 * r * cov + (vc + EPS))
    ub = (r * inv).reshape(B * B, 1).astype(jnp.bfloat16)
    ib = inv.reshape(B * B, 1).astype(jnp.bfloat16)

    s = _edge_scalars(ub, ib, P_ref, W1T_ref, R_ref, b2_ref[0, 0])
    F = s.reshape(B, B)

    rows = jax.lax.broadcasted_iota(jnp.int32, (B, B), 0) + I * B
    cols = jax.lax.broadcasted_iota(jnp.int32, (B, B), 1) + J * B
    F = jnp.where(rows == cols, 0.0, F)

    fp_ref[pl.ds(I * B, B), :] += jnp.dot(F, pj4,
                                          preferred_element_type=jnp.float32)

    @pl.when(J != I)
    def _mirror():
        ft = jax.lax.dot_general(F, pi4, (((0,), (0,)), ((), ())),
                                 preferred_element_type=jnp.float32)
        fp_ref[pl.ds(J * B, B), :] += ft

    @pl.when(p == NSTEP - 1)
    def _finalize():
        pf = pf_ref[:, :3]
        fp4 = fp_ref[...]
        rowsum = fp4[:, 3:4]
        fpos = fp4[:, :3]
        out_ref[...] = pf + (rowsum * pf - fpos) * (1.0 / (N_NODE - 1))


@functools.partial(jax.jit, static_argnames=())
def _egnn_call(pos4, P, W1T, R, b2):
    ia = jnp.asarray(np.array([p[0] for p in _PAIRS], np.int32))
    ja = jnp.asarray(np.array([p[1] for p in _PAIRS], np.int32))
    grid_spec = pltpu.PrefetchScalarGridSpec(
        num_scalar_prefetch=2,
        grid=(NSTEP,),
        in_specs=[
            pl.BlockSpec((B, 4), lambda p, ia, ja: (ia[p], 0)),
            pl.BlockSpec((B, 4), lambda p, ia, ja: (ja[p], 0)),
            pl.BlockSpec((N_NODE, 4), lambda p, ia, ja: (0, 0)),
            pl.BlockSpec((8, HIDDEN), lambda p, ia, ja: (0, 0)),
            pl.BlockSpec((HIDDEN, HIDDEN + 128), lambda p, ia, ja: (0, 0)),
            pl.BlockSpec((HIDDEN, 128), lambda p, ia, ja: (0, 0)),
            pl.BlockSpec((1, 1), lambda p, ia, ja: (0, 0)),
        ],
        out_specs=pl.BlockSpec((N_NODE, 3), lambda p, ia, ja: (0, 0)),
        scratch_shapes=[pltpu.VMEM((N_NODE, 4), jnp.float32)],
    )
    return pl.pallas_call(
        _egnn_block,
        grid_spec=grid_spec,
        out_shape=jax.ShapeDtypeStruct((N_NODE, 3), jnp.float32),
    )(ia, ja, pos4, pos4, pos4, P, W1T, R, b2)


def kernel(pos, t, W0, b0, g0, be0, W1, b1, g1, be1, W2, b2,
           senders, receivers):
    A = W0[:, 0]
    C = t * W0[:, 1] + b0
    Am = A - jnp.mean(A)
    Cm = C - jnp.mean(C)
    va = jnp.mean(Am * Am)
    cov = jnp.mean(Am * Cm)
    vc = jnp.mean(Cm * Cm)
    mom = jnp.zeros((HIDDEN,), jnp.float32).at[0].set(va).at[1].set(cov).at[2].set(vc)
    P = jnp.stack([0.5 * Am * g0, 0.5 * Cm * g0, 0.5 * be0,
                   b1 - jnp.mean(b1), 0.5 * g1, 0.5 * be1, W2[0], mom])
    W1T = W1.T
    w1m = jnp.mean(W1T, axis=1, keepdims=True)
    W1Ta = jnp.concatenate(
        [W1T, w1m, jnp.zeros((HIDDEN, 127), jnp.float32)], axis=1
    ).astype(jnp.bfloat16)
    R = jnp.concatenate(
        [jnp.full((HIDDEN, 1), 1.0 / HIDDEN, jnp.float32), W2[0][:, None],
         jnp.zeros((HIDDEN, 126), jnp.float32)], axis=1).astype(jnp.bfloat16)
    b2r = b2.reshape(1, 1)
    pos4 = jnp.concatenate([pos, jnp.ones((N_NODE, 1), jnp.float32)], axis=1)
    return _egnn_call(pos4, P, W1Ta, R, b2r)


# speedup vs baseline: 1.1143x; 1.0009x over previous
import functools

import jax
import jax.numpy as jnp
import numpy as np
from jax.experimental import pallas as pl
from jax.experimental.pallas import tpu as pltpu

N_NODE = 512
HIDDEN = 256
B = 64
NB = N_NODE // B
EPS = 1e-5

_PAIRS = [(i, j) for i in range(NB) for j in range(NB) if j >= i]
NSTEP = len(_PAIRS)


def _edge_scalars(ub, ib, P_ref, W1T_ref, R_ref, b2):
    A2 = P_ref[0:1, :].astype(jnp.bfloat16)
    C2 = P_ref[1:2, :].astype(jnp.bfloat16)
    be0 = P_ref[2:3, :].astype(jnp.bfloat16)
    b1 = P_ref[3:4, :].astype(jnp.bfloat16)
    g1 = P_ref[4:5, :].astype(jnp.bfloat16)
    be1 = P_ref[5:6, :].astype(jnp.bfloat16)
    w2 = P_ref[6:7, :].astype(jnp.bfloat16)

    a0 = ub * A2 + (ib * C2 + be0)
    x = a0 + a0 * jnp.tanh(a0)

    haug = jnp.dot(x, W1T_ref[...], preferred_element_type=jnp.float32)
    t1 = (haug[:, :HIDDEN] - haug[:, HIDDEN:HIDDEN + 1]).astype(jnp.bfloat16) + b1
    t1sq = t1 * t1
    v = jnp.dot(t1sq, R_ref[...], preferred_element_type=jnp.float32)[:, :1]
    a1 = (t1 * jax.lax.rsqrt(v + EPS).astype(jnp.bfloat16)) * g1 + be1
    y = a1 + a1 * jnp.tanh(a1)

    s = jnp.dot(y, R_ref[...], preferred_element_type=jnp.float32)[:, 1] + b2
    return s


def _egnn_block(ia_ref, ja_ref, pf_ref, P_ref, W1T_ref,
                R_ref, b2_ref, out_ref, fp_ref):
    p = pl.program_id(0)
    I = ia_ref[p]
    J = ja_ref[p]

    @pl.when(p == 0)
    def _init():
        fp_ref[...] = jnp.zeros_like(fp_ref)

    pi4 = pf_ref[pl.ds(I * B, B), :]
    pj4 = pf_ref[pl.ds(J * B, B), :]
    pi = pi4[:, :3]
    pj = pj4[:, :3]

    diff = pi[:, None, :] - pj[None, :, :]
    r = jnp.sum(diff * diff, axis=-1)

    mom = P_ref[7:8, :]
    va = mom[0, 0]
    cov = mom[0, 1]
    vc = mom[0, 2]
    inv = jax.lax.rsqrt(r * r * va + 2.0 * r * cov + (vc + EPS))
    ub = (r * inv).reshape(B * B, 1).astype(jnp.bfloat16)
    ib = inv.reshape(B * B, 1).astype(jnp.bfloat16)

    s = _edge_scalars(ub, ib, P_ref, W1T_ref, R_ref, b2_ref[0, 0])
    F = s.reshape(B, B)

    rows = jax.lax.broadcasted_iota(jnp.int32, (B, B), 0) + I * B
    cols = jax.lax.broadcasted_iota(jnp.int32, (B, B), 1) + J * B
    F = jnp.where(rows == cols, 0.0, F)

    fp_ref[pl.ds(I * B, B), :] += jnp.dot(F, pj4,
                                          preferred_element_type=jnp.float32)

    @pl.when(J != I)
    def _mirror():
        ft = jax.lax.dot_general(F, pi4, (((0,), (0,)), ((), ())),
                                 preferred_element_type=jnp.float32)
        fp_ref[pl.ds(J * B, B), :] += ft

    @pl.when(p == NSTEP - 1)
    def _finalize():
        pf = pf_ref[:, :3]
        fp4 = fp_ref[...]
        rowsum = fp4[:, 3:4]
        fpos = fp4[:, :3]
        out_ref[...] = pf + (rowsum * pf - fpos) * (1.0 / (N_NODE - 1))


@functools.partial(jax.jit, static_argnames=())
def _egnn_call(pos4, P, W1T, R, b2):
    ia = jnp.asarray(np.array([p[0] for p in _PAIRS], np.int32))
    ja = jnp.asarray(np.array([p[1] for p in _PAIRS], np.int32))
    grid_spec = pltpu.PrefetchScalarGridSpec(
        num_scalar_prefetch=2,
        grid=(NSTEP,),
        in_specs=[
            pl.BlockSpec((N_NODE, 4), lambda p, ia, ja: (0, 0)),
            pl.BlockSpec((8, HIDDEN), lambda p, ia, ja: (0, 0)),
            pl.BlockSpec((HIDDEN, HIDDEN + 128), lambda p, ia, ja: (0, 0)),
            pl.BlockSpec((HIDDEN, 128), lambda p, ia, ja: (0, 0)),
            pl.BlockSpec((1, 1), lambda p, ia, ja: (0, 0)),
        ],
        out_specs=pl.BlockSpec((N_NODE, 3), lambda p, ia, ja: (0, 0)),
        scratch_shapes=[pltpu.VMEM((N_NODE, 4), jnp.float32)],
    )
    return pl.pallas_call(
        _egnn_block,
        grid_spec=grid_spec,
        out_shape=jax.ShapeDtypeStruct((N_NODE, 3), jnp.float32),
    )(ia, ja, pos4, P, W1T, R, b2)


def kernel(pos, t, W0, b0, g0, be0, W1, b1, g1, be1, W2, b2,
           senders, receivers):
    A = W0[:, 0]
    C = t * W0[:, 1] + b0
    Am = A - jnp.mean(A)
    Cm = C - jnp.mean(C)
    va = jnp.mean(Am * Am)
    cov = jnp.mean(Am * Cm)
    vc = jnp.mean(Cm * Cm)
    mom = jnp.zeros((HIDDEN,), jnp.float32).at[0].set(va).at[1].set(cov).at[2].set(vc)
    P = jnp.stack([0.5 * Am * g0, 0.5 * Cm * g0, 0.5 * be0,
                   b1 - jnp.mean(b1), 0.5 * g1, 0.5 * be1, W2[0], mom])
    W1T = W1.T
    w1m = jnp.mean(W1T, axis=1, keepdims=True)
    W1Ta = jnp.concatenate(
        [W1T, w1m, jnp.zeros((HIDDEN, 127), jnp.float32)], axis=1
    ).astype(jnp.bfloat16)
    R = jnp.concatenate(
        [jnp.full((HIDDEN, 1), 1.0 / HIDDEN, jnp.float32), W2[0][:, None],
         jnp.zeros((HIDDEN, 126), jnp.float32)], axis=1).astype(jnp.bfloat16)
    b2r = b2.reshape(1, 1)
    pos4 = jnp.concatenate([pos, jnp.ones((N_NODE, 1), jnp.float32)], axis=1)
    return _egnn_call(pos4, P, W1Ta, R, b2r)
